# baseline (speedup 1.0000x reference)
import functools

import jax
import jax.numpy as jnp
from jax import lax
from jax.experimental import pallas as pl
from jax.experimental.pallas import tpu as pltpu
from jax.experimental.pallas import tpu_sc as plsc

B = 16
L = 4096
T = 32768
D = 256

NC = 2
NS = 16
NW = NC * NS
RW = (B * L) // NW
CH = 64
NB = 6
NCH = RW // CH
VPC = CH // 16
DV = D // 16
CU_PAD = 32


def _sc_body(feat_hbm, cu_hbm, out_hbm, cu_v, zero_v, idx, buf, sg, sw):
    wid = lax.axis_index("s") * NC + lax.axis_index("c")
    lanes = lax.iota(jnp.int32, 16)

    pltpu.sync_copy(cu_hbm, cu_v)

    b = wid // (L // RW)
    p0 = (wid % (L // RW)) * RW
    r0 = wid * RW

    cu_pair = cu_v[pl.ds(b, 16)]
    start = cu_pair[0]
    end = cu_pair[1]
    nv = jnp.clip(jnp.minimum(end - start, L) - p0, 0, RW)
    nvc = [jnp.clip(nv - c * CH, 0, CH) for c in range(NCH)]

    zf = jnp.zeros((16,), jnp.float32)

    def _zrow(r, carry):
        for d in range(DV):
            zero_v[r, pl.ds(d * 16, 16)] = zf
        return carry

    lax.fori_loop(0, CH, _zrow, 0)

    def start_gather(c):
        i = c % NB

        @pl.when(nvc[c] > 0)
        def _():
            src0 = start + p0 + c * CH
            for v in range(VPC):
                idx[i][pl.ds(v * 16, 16)] = jnp.minimum(
                    src0 + v * 16 + lanes, T - 1)
            pltpu.async_copy(feat_hbm.at[idx[i]], buf[i], sg[i])

    def wait_gather(c):
        i = c % NB

        @pl.when(nvc[c] > 0)
        def _():
            pltpu.make_async_copy(feat_hbm.at[idx[i]], buf[i], sg[i]).wait()

    def start_write(c):
        i = c % NB
        dst = out_hbm.at[pl.ds(r0 + c * CH, CH)]

        @pl.when(nvc[c] > 0)
        def _():
            def _ztail(j, carry):
                for d in range(DV):
                    buf[i][j, pl.ds(d * 16, 16)] = zf
                return carry

            lax.fori_loop(nvc[c], CH, _ztail, 0)
            pltpu.async_copy(buf[i], dst, sw[i])

        @pl.when(nvc[c] == 0)
        def _():
            pltpu.async_copy(zero_v, dst, sw[i])

    def wait_write(c):
        i = c % NB
        dst = out_hbm.at[pl.ds(r0 + c * CH, CH)]
        pltpu.make_async_copy(buf[i], dst, sw[i]).wait()

    for c in range(min(NB - 1, NCH)):
        start_gather(c)
    for c in range(NCH):
        wait_gather(c)
        start_write(c)
        nxt = c + NB - 1
        if nxt < NCH:
            if nxt - NB >= 0:
                wait_write(nxt - NB)
            start_gather(nxt)
    for c in range(max(0, NCH - NB), NCH):
        wait_write(c)


@functools.partial(
    pl.kernel,
    mesh=plsc.VectorSubcoreMesh(core_axis_name="c", subcore_axis_name="s"),
    out_type=jax.ShapeDtypeStruct((B * L, D), jnp.float32),
    scratch_types=(
        [pltpu.VMEM((CU_PAD,), jnp.int32), pltpu.VMEM((CH, D), jnp.float32)]
        + [pltpu.VMEM((CH,), jnp.int32) for _ in range(NB)]
        + [pltpu.VMEM((CH, D), jnp.float32) for _ in range(NB)]
        + [pltpu.SemaphoreType.DMA for _ in range(2 * NB)]
    ),
)
def _sc_scatter(feat_hbm, cu_hbm, out_hbm, cu_v, zero_v, *rest):
    idx = rest[:NB]
    buf = rest[NB:2 * NB]
    sg = rest[2 * NB:3 * NB]
    sw = rest[3 * NB:4 * NB]
    _sc_body(feat_hbm, cu_hbm, out_hbm, cu_v, zero_v, idx, buf, sg, sw)


@jax.jit
def kernel(input_features, cu_seqlens):
    cu_pad = jnp.concatenate(
        [
            cu_seqlens.astype(jnp.int32),
            jnp.full((CU_PAD - (B + 1),), T, dtype=jnp.int32),
        ]
    )
    out = _sc_scatter(input_features, cu_pad)
    return out.reshape(B, L, D)


# speedup vs baseline: 1.0112x; 1.0112x over previous
import functools

import jax
import jax.numpy as jnp
from jax import lax
from jax.experimental import pallas as pl
from jax.experimental.pallas import tpu as pltpu
from jax.experimental.pallas import tpu_sc as plsc

B = 16
L = 4096
T = 32768
D = 256

NC = 2
NS = 16
NW = NC * NS
CH = 128
NB = 3
PCB = L // CH
NCH = (B * L) // (CH * NW)
STRIDE = 17
VPC = CH // 16
DV = D // 16
CU_PAD = 32


def _sc_body(feat_hbm, cu_hbm, out_hbm, cu_v, zero_v, idx, buf, sg, sw):
    wid = lax.axis_index("s") * NC + lax.axis_index("c")
    lanes = lax.iota(jnp.int32, 16)

    pltpu.sync_copy(cu_hbm, cu_v)

    src0s, nvcs, outs = [], [], []
    for k in range(NCH):
        cu_pair = cu_v[pl.ds(k, 16)]
        start = cu_pair[0]
        end = cu_pair[1]
        pos = ((wid + STRIDE * k) % PCB) * CH
        src0s.append(start + pos)
        nvcs.append(jnp.clip(jnp.minimum(end - start, L) - pos, 0, CH))
        outs.append(k * L + pos)

    zf = jnp.zeros((16,), jnp.float32)

    def _zrow(r, carry):
        for d in range(DV):
            zero_v[r, pl.ds(d * 16, 16)] = zf
        return carry

    lax.fori_loop(0, CH // 2, _zrow, 0)

    def start_gather(c):
        i = c % NB

        @pl.when(nvcs[c] > 0)
        def _():
            for v in range(VPC):
                idx[i][pl.ds(v * 16, 16)] = jnp.minimum(
                    src0s[c] + v * 16 + lanes, T - 1)
            pltpu.async_copy(feat_hbm.at[idx[i]], buf[i], sg[i])

    def wait_gather(c):
        i = c % NB

        @pl.when(nvcs[c] > 0)
        def _():
            pltpu.make_async_copy(feat_hbm.at[idx[i]], buf[i], sg[i]).wait()

    def start_write(c):
        i = c % NB
        dst = out_hbm.at[pl.ds(outs[c], CH)]

        @pl.when(nvcs[c] > 0)
        def _():
            def _ztail(j, carry):
                for d in range(DV):
                    buf[i][j, pl.ds(d * 16, 16)] = zf
                return carry

            lax.fori_loop(nvcs[c], CH, _ztail, 0)
            pltpu.async_copy(buf[i], dst, sw[i])

        @pl.when(nvcs[c] == 0)
        def _():
            pltpu.async_copy(
                zero_v, out_hbm.at[pl.ds(outs[c], CH // 2)], sw[i])
            pltpu.async_copy(
                zero_v, out_hbm.at[pl.ds(outs[c] + CH // 2, CH // 2)], sw[i])

    def wait_write(c):
        i = c % NB
        dst = out_hbm.at[pl.ds(outs[c], CH)]
        pltpu.make_async_copy(buf[i], dst, sw[i]).wait()

    for c in range(min(NB - 1, NCH)):
        start_gather(c)
    for c in range(NCH):
        wait_gather(c)
        start_write(c)
        nxt = c + NB - 1
        if nxt < NCH:
            if nxt - NB >= 0:
                wait_write(nxt - NB)
            start_gather(nxt)
    for c in range(max(0, NCH - NB), NCH):
        wait_write(c)


@functools.partial(
    pl.kernel,
    mesh=plsc.VectorSubcoreMesh(core_axis_name="c", subcore_axis_name="s"),
    out_type=jax.ShapeDtypeStruct((B * L, D), jnp.float32),
    scratch_types=(
        [pltpu.VMEM((CU_PAD,), jnp.int32),
         pltpu.VMEM((CH // 2, D), jnp.float32)]
        + [pltpu.VMEM((CH,), jnp.int32) for _ in range(NB)]
        + [pltpu.VMEM((CH, D), jnp.float32) for _ in range(NB)]
        + [pltpu.SemaphoreType.DMA for _ in range(2 * NB)]
    ),
)
def _sc_scatter(feat_hbm, cu_hbm, out_hbm, cu_v, zero_v, *rest):
    idx = rest[:NB]
    buf = rest[NB:2 * NB]
    sg = rest[2 * NB:3 * NB]
    sw = rest[3 * NB:4 * NB]
    _sc_body(feat_hbm, cu_hbm, out_hbm, cu_v, zero_v, idx, buf, sg, sw)


@jax.jit
def kernel(input_features, cu_seqlens):
    cu_pad = jnp.concatenate(
        [
            cu_seqlens.astype(jnp.int32),
            jnp.full((CU_PAD - (B + 1),), T, dtype=jnp.int32),
        ]
    )
    out = _sc_scatter(input_features, cu_pad)
    return out.reshape(B, L, D)
